# baseline (speedup 1.0000x reference)
import jax
import jax.numpy as jnp
from jax import lax
from jax.experimental import pallas as pl
from jax.experimental.pallas import tpu as pltpu
from jax.experimental.pallas import tpu_sc as plsc

H = 24
W = 24
HW = H * W
NPLANES = 64 * 384
NTILES = 32
PLANES_PER_TILE = NPLANES // NTILES
CHUNK = 64
NCHUNKS = PLANES_PER_TILE // CHUNK
NGROUPS = CHUNK // 16
NV = HW // 16
HALF = 3


def _tile_body(x_hbm, t_hbm, o_hbm, tbuf, buf, sem):
    del sem
    wid = lax.axis_index("s") * 2 + lax.axis_index("c")
    tile_base = wid * PLANES_PER_TILE
    pltpu.sync_copy(t_hbm.at[pl.ds(tile_base, PLANES_PER_TILE)], tbuf)

    lane = lax.iota(jnp.int32, 16)
    krs, kcs = [], []
    for v in range(3):
        k = lane + v * 16
        kr = k // 6
        krs.append(kr)
        kcs.append(k - kr * 6)
    zero16 = jnp.zeros((16,), jnp.float32)
    neginf = jnp.full((16,), -3.4e38, jnp.float32)

    def group_body(ci, g):
        base0 = g * (16 * HW)
        idx0 = base0 + lane * HW

        def step(j, carry):
            m, jb = carry
            v = plsc.load_gather(buf, [idx0 + j])
            gt = v > m
            return jnp.where(gt, v, m), jnp.where(gt, j, jb)

        _, jb = lax.fori_loop(0, HW, step,
                              (neginf, jnp.zeros((16,), jnp.int32)),
                              unroll=8)

        hh = jb // W
        ww = jb - hh * W
        h1 = jnp.clip(hh - HALF, 0, H - 1)
        h2 = jnp.clip(hh + HALF, 0, H - 1)
        w1 = jnp.clip(ww - HALF, 0, W - 1)
        w2 = jnp.clip(ww + HALF, 0, W - 1)
        dh = h2 - h1
        dw = w2 - w1
        area = (dh * dw).astype(jnp.float32)
        lam = jnp.float32(HW) / (jnp.float32(HW) - area)

        tv = tbuf[pl.ds(ci * CHUNK + g * 16, 16)]

        for p in range(16):
            tp = tv[p]

            @pl.when(tp != 0.0)
            def _(p=p):
                lamp = lam[p]
                pb = base0 + p * HW
                for c in range(NV):
                    sl = pl.ds(pb + c * 16, 16)
                    buf[sl] = buf[sl] * lamp
                h1p = h1[p]
                w1p = w1[p]
                dhp = dh[p]
                dwp = dw[p]
                for v in range(3):
                    bidx = pb + (h1p + krs[v]) * W + (w1p + kcs[v])
                    msk = (krs[v] < dhp) & (kcs[v] < dwp)
                    plsc.store_scatter(buf, [bidx], zero16, mask=msk)

    def chunk_body(ci, carry):
        base_el = (tile_base + ci * CHUNK) * HW
        pltpu.sync_copy(x_hbm.at[pl.ds(base_el, CHUNK * HW)], buf)
        lax.fori_loop(0, NGROUPS, lambda g, c: (group_body(ci, g), c)[1], 0,
                      unroll=False)
        pltpu.sync_copy(buf, o_hbm.at[pl.ds(base_el, CHUNK * HW)])
        return carry

    lax.fori_loop(0, NCHUNKS, chunk_body, 0, unroll=False)


@jax.jit
def kernel(x, T):
    batch, channel, h, w = x.shape
    xf = x.reshape(-1)
    tf = T.reshape(-1)
    mesh = plsc.VectorSubcoreMesh(core_axis_name="c", subcore_axis_name="s")
    run = pl.kernel(
        _tile_body,
        out_type=jax.ShapeDtypeStruct((NPLANES * HW,), jnp.float32),
        mesh=mesh,
        scratch_types=[
            pltpu.VMEM((PLANES_PER_TILE,), jnp.float32),
            pltpu.VMEM((CHUNK * HW,), jnp.float32),
            pltpu.SemaphoreType.DMA,
        ],
        compiler_params=pltpu.CompilerParams(needs_layout_passes=False),
    )
    out = run(xf, tf)
    return out.reshape(batch, channel, h, w)
